# Initial kernel scaffold; baseline (speedup 1.0000x reference)
import jax
import jax.numpy as jnp
from jax.experimental import pallas as pl


def kernel(token_ids, lengths, token_table, length_table):
    raise NotImplementedError("write your pallas kernel here")


# speedup vs baseline: 8.4251x; 8.4251x over previous
import functools
import math

import numpy as np
import jax
import jax.numpy as jnp
from jax import lax
from jax.experimental import pallas as pl
from jax.experimental.pallas import tpu as pltpu
from jax.experimental.pallas import tpu_sc as plsc

_NC = 2
_NS = 16
_LANES = 16


def _pos_encoding(max_len, d_model):
    position = np.arange(max_len, dtype=np.float32)[:, None]
    div_term = np.exp(
        np.arange(0, d_model, 2).astype(np.float32) * (-math.log(10000.0) / d_model)
    )
    pe = np.zeros((max_len, d_model), dtype=np.float32)
    pe[:, 0::2] = np.sin(position * div_term)
    pe[:, 1::2] = np.cos(position * div_term)
    return pe


def kernel(token_ids, lengths, token_table, length_table):
    B, S = token_ids.shape
    _, D = token_table.shape
    scale = float(math.sqrt(D))
    pe = jnp.asarray(_pos_encoding(S, D))

    NW = _NC * _NS
    assert B % NW == 0 and D % _LANES == 0
    BW = B // NW
    NCH = D // _LANES

    mesh = plsc.VectorSubcoreMesh(
        core_axis_name="c", subcore_axis_name="s",
        num_cores=_NC, num_subcores=_NS,
    )

    @functools.partial(
        pl.kernel,
        out_type=jax.ShapeDtypeStruct((B, S, D), jnp.float32),
        mesh=mesh,
        scratch_types=[
            pltpu.VMEM((BW, S), jnp.int32),
            pltpu.VMEM((BW,), jnp.int32),
            pltpu.VMEM((BW, D), jnp.float32),
            pltpu.VMEM((S, D), jnp.float32),
            pltpu.VMEM((S, D), jnp.float32),
            pltpu.VMEM((S, D), jnp.float32),
            pltpu.VMEM((S, D), jnp.float32),
            pltpu.VMEM((S, D), jnp.float32),
            pltpu.SemaphoreType.DMA,
            pltpu.SemaphoreType.DMA,
            pltpu.SemaphoreType.DMA,
            pltpu.SemaphoreType.DMA,
        ],
    )
    def run(ids_hbm, len_hbm, tab_hbm, ltab_hbm, pe_hbm, out_hbm,
            ids_v, lidx_v, lrows_v, pe_v, gbuf0, gbuf1, obuf0, obuf1,
            gsem0, gsem1, osem0, osem1):
        wid = lax.axis_index("s") * _NC + lax.axis_index("c")
        base = wid * BW

        pltpu.sync_copy(ids_hbm.at[pl.ds(base, BW)], ids_v)
        pltpu.sync_copy(len_hbm.at[pl.ds(base, BW)], lidx_v)
        pltpu.sync_copy(pe_hbm, pe_v)
        pltpu.async_copy(ltab_hbm.at[lidx_v], lrows_v, gsem0).wait()

        gbufs = (gbuf0, gbuf1)
        obufs = (obuf0, obuf1)
        gsems = (gsem0, gsem1)
        osems = (osem0, osem1)

        pltpu.async_copy(tab_hbm.at[ids_v.at[0]], gbufs[0], gsems[0])

        @pl.loop(0, BW // 2)
        def _outer(jj):
            for b in range(2):
                j = jj * 2 + b
                k = b
                nk = 1 - b

                @pl.when(j + 1 < BW)
                def _():
                    pltpu.async_copy(
                        tab_hbm.at[ids_v.at[j + 1]], gbufs[nk], gsems[nk])

                pltpu.make_async_copy(
                    tab_hbm.at[ids_v.at[j]], gbufs[k], gsems[k]).wait()

                @pl.when(j >= 2)
                def _():
                    pltpu.make_async_copy(
                        obufs[k], out_hbm.at[base + j - 2], osems[k]).wait()

                lvecs = [lrows_v[j, pl.ds(c * _LANES, _LANES)]
                         for c in range(NCH)]

                @pl.loop(0, S)
                def _srow(s):
                    for c in range(NCH):
                        sl = pl.ds(c * _LANES, _LANES)
                        g = gbufs[k][s, sl]
                        obufs[k][s, sl] = g * scale + pe_v[s, sl] + lvecs[c]

                pltpu.async_copy(obufs[k], out_hbm.at[base + j], osems[k])

        pltpu.make_async_copy(obufs[0], out_hbm.at[base + BW - 2], osems[0]).wait()
        pltpu.make_async_copy(obufs[1], out_hbm.at[base + BW - 1], osems[1]).wait()

    return run(token_ids, lengths, token_table, length_table, pe)
